# baseline (speedup 1.0000x reference)
import functools

import jax
import jax.numpy as jnp
from jax.experimental import pallas as pl
from jax.experimental.pallas import tpu as pltpu

_NUM_EMB = 1024
_DIM = 64
_ROWS = 64 * 576
_BLK = 1024
_NBLK = _ROWS // _BLK


def _vq_block_kernel(x_ref, w_ref, x2_ref, w2_ref, qst_ref, idx_ref, acc_ref):
    x = x_ref[...]
    w = w_ref[...]
    s = jax.lax.dot_general(x, w, (((1,), (1,)), ((), ())),
                            preferred_element_type=jnp.float32)
    d = (x2_ref[...] + w2_ref[...]) - 2.0 * s
    m = jnp.min(d, axis=1, keepdims=True)
    lane = jax.lax.broadcasted_iota(jnp.int32, (_BLK, _NUM_EMB), 1)
    idx = jnp.min(jnp.where(d == m, lane, _NUM_EMB), axis=1)
    enc = (lane == idx[:, None]).astype(jnp.bfloat16)
    q = jax.lax.dot_general(enc, w.astype(jnp.bfloat16),
                            (((1,), (0,)), ((), ())),
                            preferred_element_type=jnp.float32)
    qst_ref[...] = x + (q - x)
    idx_ref[...] = idx[:, None]
    part = jnp.sum((q - x) ** 2)

    @pl.when(pl.program_id(0) == 0)
    def _init():
        acc_ref[...] = jnp.zeros_like(acc_ref)

    acc_ref[...] += part


@functools.partial(jax.jit, static_argnames=())
def kernel(inputs, W):
    input_shape = inputs.shape
    flat = inputs.reshape(-1, _DIM)
    x2 = jnp.sum(flat ** 2, axis=1, keepdims=True)
    w2 = jnp.sum(W ** 2, axis=1).reshape(1, _NUM_EMB)

    qst, idx, acc = pl.pallas_call(
        _vq_block_kernel,
        grid=(_NBLK,),
        in_specs=[
            pl.BlockSpec((_BLK, _DIM), lambda i: (i, 0)),
            pl.BlockSpec((_NUM_EMB, _DIM), lambda i: (0, 0)),
            pl.BlockSpec((_BLK, 1), lambda i: (i, 0)),
            pl.BlockSpec((1, _NUM_EMB), lambda i: (0, 0)),
        ],
        out_specs=[
            pl.BlockSpec((_BLK, _DIM), lambda i: (i, 0)),
            pl.BlockSpec((_BLK, 1), lambda i: (i, 0)),
            pl.BlockSpec((1, 1), lambda i: (0, 0)),
        ],
        out_shape=[
            jax.ShapeDtypeStruct((_ROWS, _DIM), jnp.float32),
            jax.ShapeDtypeStruct((_ROWS, 1), jnp.int32),
            jax.ShapeDtypeStruct((1, 1), jnp.float32),
        ],
        compiler_params=pltpu.CompilerParams(
            dimension_semantics=("arbitrary",),
        ),
    )(flat, W, x2, w2)

    mse = acc[0, 0] / jnp.float32(flat.size)
    loss = mse + 0.25 * mse
    return (loss, qst.reshape(input_shape), idx)


# speedup vs baseline: 1.0948x; 1.0948x over previous
import functools

import jax
import jax.numpy as jnp
from jax.experimental import pallas as pl
from jax.experimental.pallas import tpu as pltpu

_NUM_EMB = 1024
_DIM = 64
_ROWS = 64 * 576
_BLK = 1024
_NBLK = _ROWS // _BLK


def _vq_block_kernel(x_ref, w_ref, w2_ref, qst_ref, idx_ref, acc_ref):
    x = x_ref[...]
    w = w_ref[...]
    s = jax.lax.dot_general(x, w, (((1,), (1,)), ((), ())),
                            preferred_element_type=jnp.float32)
    x2 = jnp.sum(x * x, axis=1, keepdims=True)
    d = (x2 + w2_ref[...]) - 2.0 * s
    m = jnp.min(d, axis=1, keepdims=True)
    lane = jax.lax.broadcasted_iota(jnp.int32, (_BLK, _NUM_EMB), 1)
    idx = jnp.min(jnp.where(d == m, lane, _NUM_EMB), axis=1)
    enc = (lane == idx[:, None]).astype(jnp.bfloat16)
    q = jax.lax.dot_general(enc, w.astype(jnp.bfloat16),
                            (((1,), (0,)), ((), ())),
                            preferred_element_type=jnp.float32)
    qst_ref[...] = x + (q - x)
    idx_ref[...] = idx
    part = jnp.sum((q - x) ** 2)

    @pl.when(pl.program_id(0) == 0)
    def _init():
        acc_ref[...] = jnp.zeros_like(acc_ref)

    acc_ref[...] += part


@functools.partial(jax.jit, static_argnames=())
def kernel(inputs, W):
    input_shape = inputs.shape
    flat = inputs.reshape(-1, _DIM)
    w2 = jnp.sum(W ** 2, axis=1).reshape(1, _NUM_EMB)

    qst, idx, acc = pl.pallas_call(
        _vq_block_kernel,
        grid=(_NBLK,),
        in_specs=[
            pl.BlockSpec((_BLK, _DIM), lambda i: (i, 0)),
            pl.BlockSpec((_NUM_EMB, _DIM), lambda i: (0, 0)),
            pl.BlockSpec((1, _NUM_EMB), lambda i: (0, 0)),
        ],
        out_specs=[
            pl.BlockSpec((_BLK, _DIM), lambda i: (i, 0)),
            pl.BlockSpec((_BLK,), lambda i: (i,)),
            pl.BlockSpec((1, 1), lambda i: (0, 0)),
        ],
        out_shape=[
            jax.ShapeDtypeStruct((_ROWS, _DIM), jnp.float32),
            jax.ShapeDtypeStruct((_ROWS,), jnp.int32),
            jax.ShapeDtypeStruct((1, 1), jnp.float32),
        ],
        compiler_params=pltpu.CompilerParams(
            dimension_semantics=("arbitrary",),
        ),
    )(flat, W, w2)

    mse = acc[0, 0] / jnp.float32(flat.size)
    loss = mse + 0.25 * mse
    return (loss, qst.reshape(input_shape), idx[:, None])
